# baseline (speedup 1.0000x reference)
import jax
import jax.numpy as jnp
from jax import lax
from jax.experimental import pallas as pl
from jax.experimental.pallas import tpu as pltpu
from jax.experimental.pallas import tpu_sc as plsc

CODEBOOK_NUM = 2
CENTROIDS = 256
BLOCK = 64
ROWS = 4096
COLS = 4096
NUMEL = ROWS * COLS
NVEC = NUMEL // 4
NBLOCKS = NUMEL // BLOCK

NUM_WORKERS = 32
BAND_ROWS = 8
NBANDS = ROWS // BAND_ROWS
BANDS_W = NBANDS // NUM_WORKERS
VC = BAND_ROWS * COLS // 4
ITERS = VC // 16
NPH = 2


def _vperm(vec, idx):
    dnums = lax.GatherDimensionNumbers(
        offset_dims=(), collapsed_slice_dims=(0,), start_index_map=(0,))
    return lax.gather(vec, idx[:, None], dimension_numbers=dnums,
                      slice_sizes=(1,),
                      mode=lax.GatherScatterMode.PROMISE_IN_BOUNDS)


def _body(codes_hbm, table_hbm, scales_hbm, out_hbm, table_v, *scratch):
    nc = plsc.get_sparse_core_info().num_cores
    wid = lax.axis_index("s") * nc + lax.axis_index("c")

    cb_row = wid // (NUM_WORKERS // CODEBOOK_NUM)
    cb_off = (cb_row * CENTROIDS * 4).astype(jnp.int32)
    lane = lax.iota(jnp.int32, 16)
    laneoff = (lane & 3) + cb_off
    rep = [(lane >> 2) + 4 * r for r in range(4)]

    cbufs = scratch[0:NPH]
    sbufs = scratch[NPH:2 * NPH]
    obufs = scratch[2 * NPH:3 * NPH]
    in_sems = scratch[3 * NPH:4 * NPH]
    out_sems = scratch[4 * NPH:5 * NPH]

    def start_in(k, cbuf, sbuf, sem):
        b = wid * BANDS_W + k
        col0 = pl.multiple_of((b % (NBANDS // CODEBOOK_NUM)) * VC, VC)
        pltpu.async_copy(codes_hbm.at[cb_row, pl.ds(col0, VC)], cbuf, sem)
        pltpu.async_copy(
            scales_hbm.at[pl.ds(pl.multiple_of(b * (VC // 16), VC // 16),
                                VC // 16)],
            sbuf, sem)

    def out_slice(b):
        return out_hbm.at[pl.ds(pl.multiple_of(b * BAND_ROWS, BAND_ROWS),
                                BAND_ROWS), :]

    def compute(cbuf, sbuf, obuf):
        @plsc.parallel_loop(0, ITERS, 1, unroll=16)
        def vec_body(t):
            c = cbuf[pl.ds(t * 16, 16)]
            svec = plsc.load_gather(sbuf, [jnp.full((16,), t, jnp.int32)])
            c4 = c * 4
            row = t >> 6
            base = (t & 63) * 64
            for r in range(4):
                crep = _vperm(c4, rep[r])
                vals = plsc.load_gather(table_v, [crep + laneoff])
                obuf[row, pl.ds(base + r * 16, 16)] = vals * svec

    start_in(0, cbufs[0], sbufs[0], in_sems[0])
    start_in(1, cbufs[1], sbufs[1], in_sems[1])
    pltpu.sync_copy(table_hbm, table_v)

    nph = len(obufs)

    def step(m, _):
        for phase in range(nph):
            k = nph * m + phase
            cbuf, sbuf, obuf = cbufs[phase], sbufs[phase], obufs[phase]
            in_sem, out_sem = in_sems[phase], out_sems[phase]
            b = wid * BANDS_W + k

            pltpu.make_async_copy(codes_hbm.at[cb_row, pl.ds(0, VC)],
                                  cbuf, in_sem).wait()
            pltpu.make_async_copy(scales_hbm.at[pl.ds(0, VC // 16)],
                                  sbuf, in_sem).wait()

            @pl.when(m > 0)
            def _():
                pltpu.make_async_copy(obuf, out_slice(b), out_sem).wait()

            compute(cbuf, sbuf, obuf)
            pltpu.async_copy(obuf, out_slice(b), out_sem)

            p2 = (phase + 2) % nph

            @pl.when(k + 2 < BANDS_W)
            def _():
                start_in(k + 2, cbufs[p2], sbufs[p2], in_sems[p2])
        return 0

    lax.fori_loop(0, BANDS_W // nph, step, 0)
    for phase in range(nph):
        pltpu.make_async_copy(obufs[phase], out_slice(0),
                              out_sems[phase]).wait()


@jax.jit
def _dequant(codes, table_flat, scales_flat):
    mesh = plsc.VectorSubcoreMesh(core_axis_name="c", subcore_axis_name="s")
    run = pl.kernel(
        _body,
        out_type=jax.ShapeDtypeStruct((ROWS, COLS), jnp.float32),
        mesh=mesh,
        scratch_types=(
            [pltpu.VMEM((CODEBOOK_NUM * CENTROIDS * 4,), jnp.float32)]
            + [pltpu.VMEM((VC,), jnp.int32)] * NPH
            + [pltpu.VMEM((VC // 16,), jnp.float32)] * NPH
            + [pltpu.VMEM((BAND_ROWS, COLS), jnp.float32)] * NPH
            + [pltpu.SemaphoreType.DMA] * (2 * NPH)
        ),
        compiler_params=pltpu.CompilerParams(needs_layout_passes=False,
                                             use_tc_tiling_on_sc=True),
    )
    return run(codes, table_flat, scales_flat)


def kernel(codes, codebooks, scales):
    table_flat = codebooks.reshape(CODEBOOK_NUM * CENTROIDS * 4)
    scales_flat = scales.reshape(NBLOCKS)
    return _dequant(codes, table_flat, scales_flat)


# speedup vs baseline: 1.0557x; 1.0557x over previous
import jax
import jax.numpy as jnp
from jax import lax
from jax.experimental import pallas as pl
from jax.experimental.pallas import tpu as pltpu
from jax.experimental.pallas import tpu_sc as plsc

CODEBOOK_NUM = 2
CENTROIDS = 256
BLOCK = 64
ROWS = 4096
COLS = 4096
NUMEL = ROWS * COLS
NVEC = NUMEL // 4
NBLOCKS = NUMEL // BLOCK

NUM_WORKERS = 32
BAND_ROWS = 8
NBANDS = ROWS // BAND_ROWS
BANDS_W = NBANDS // NUM_WORKERS
VC = BAND_ROWS * COLS // 4
ITERS = VC // 16
NPH = 2


def _vperm(vec, idx):
    dnums = lax.GatherDimensionNumbers(
        offset_dims=(), collapsed_slice_dims=(0,), start_index_map=(0,))
    return lax.gather(vec, idx[:, None], dimension_numbers=dnums,
                      slice_sizes=(1,),
                      mode=lax.GatherScatterMode.PROMISE_IN_BOUNDS)


def _body(codes_hbm, table_hbm, scales_hbm, out_hbm, table_v, *scratch):
    nc = plsc.get_sparse_core_info().num_cores
    wid = lax.axis_index("s") * nc + lax.axis_index("c")

    cb_row = wid // (NUM_WORKERS // CODEBOOK_NUM)
    cb_off = (cb_row * CENTROIDS * 4).astype(jnp.int32)
    lane = lax.iota(jnp.int32, 16)
    laneoff = (lane & 3) + cb_off
    rep = [(lane >> 2) + 4 * r for r in range(4)]

    cbufs = scratch[0:NPH]
    sbufs = scratch[NPH:2 * NPH]
    obufs = scratch[2 * NPH:3 * NPH]
    in_sems = scratch[3 * NPH:4 * NPH]
    out_sems = scratch[4 * NPH:5 * NPH]

    def start_in(k, cbuf, sbuf, sem):
        b = wid * BANDS_W + k
        col0 = pl.multiple_of((b % (NBANDS // CODEBOOK_NUM)) * VC, VC)
        pltpu.async_copy(codes_hbm.at[cb_row, pl.ds(col0, VC)], cbuf, sem)
        pltpu.async_copy(
            scales_hbm.at[pl.ds(pl.multiple_of(b * (VC // 16), VC // 16),
                                VC // 16)],
            sbuf, sem)

    def out_slice(b):
        return out_hbm.at[pl.ds(pl.multiple_of(b * BAND_ROWS, BAND_ROWS),
                                BAND_ROWS), :]

    def compute(cbuf, sbuf, obuf):
        @plsc.parallel_loop(0, ITERS, 1, unroll=4)
        def vec_body(t):
            c = cbuf[pl.ds(t * 16, 16)]
            svec = plsc.load_gather(sbuf, [jnp.full((16,), t, jnp.int32)])
            c4 = c * 4
            row = t >> 6
            base = (t & 63) * 64
            for r in range(4):
                crep = _vperm(c4, rep[r])
                vals = plsc.load_gather(table_v, [crep + laneoff])
                obuf[row, pl.ds(base + r * 16, 16)] = vals * svec

    start_in(0, cbufs[0], sbufs[0], in_sems[0])
    start_in(1, cbufs[1], sbufs[1], in_sems[1])
    pltpu.sync_copy(table_hbm, table_v)

    nph = len(obufs)

    def step(m, _):
        for phase in range(nph):
            k = nph * m + phase
            cbuf, sbuf, obuf = cbufs[phase], sbufs[phase], obufs[phase]
            in_sem, out_sem = in_sems[phase], out_sems[phase]
            b = wid * BANDS_W + k

            pltpu.make_async_copy(codes_hbm.at[cb_row, pl.ds(0, VC)],
                                  cbuf, in_sem).wait()
            pltpu.make_async_copy(scales_hbm.at[pl.ds(0, VC // 16)],
                                  sbuf, in_sem).wait()

            @pl.when(m > 0)
            def _():
                pltpu.make_async_copy(obuf, out_slice(b), out_sem).wait()

            compute(cbuf, sbuf, obuf)
            pltpu.async_copy(obuf, out_slice(b), out_sem)

            p2 = (phase + 2) % nph

            @pl.when(k + 2 < BANDS_W)
            def _():
                start_in(k + 2, cbufs[p2], sbufs[p2], in_sems[p2])
        return 0

    lax.fori_loop(0, BANDS_W // nph, step, 0)
    for phase in range(nph):
        pltpu.make_async_copy(obufs[phase], out_slice(0),
                              out_sems[phase]).wait()


@jax.jit
def _dequant(codes, table_flat, scales_flat):
    mesh = plsc.VectorSubcoreMesh(core_axis_name="c", subcore_axis_name="s")
    run = pl.kernel(
        _body,
        out_type=jax.ShapeDtypeStruct((ROWS, COLS), jnp.float32),
        mesh=mesh,
        scratch_types=(
            [pltpu.VMEM((CODEBOOK_NUM * CENTROIDS * 4,), jnp.float32)]
            + [pltpu.VMEM((VC,), jnp.int32)] * NPH
            + [pltpu.VMEM((VC // 16,), jnp.float32)] * NPH
            + [pltpu.VMEM((BAND_ROWS, COLS), jnp.float32)] * NPH
            + [pltpu.SemaphoreType.DMA] * (2 * NPH)
        ),
        compiler_params=pltpu.CompilerParams(needs_layout_passes=False,
                                             use_tc_tiling_on_sc=True),
    )
    return run(codes, table_flat, scales_flat)


def kernel(codes, codebooks, scales):
    table_flat = codebooks.reshape(CODEBOOK_NUM * CENTROIDS * 4)
    scales_flat = scales.reshape(NBLOCKS)
    return _dequant(codes, table_flat, scales_flat)
